# baseline (speedup 1.0000x reference)
import jax
import jax.numpy as jnp
from jax.experimental import pallas as pl
from jax.experimental.pallas import tpu as pltpu

_N_SPLIT = 4


def _dma_copy(t_ref, o_ref, sems):
    rows = o_ref.shape[0]
    chunk = rows // _N_SPLIT
    copies = [
        pltpu.make_async_copy(
            t_ref.at[pl.ds(i * chunk, chunk), :],
            o_ref.at[pl.ds(i * chunk, chunk), :],
            sems.at[i],
        )
        for i in range(_N_SPLIT)
    ]
    for c in copies:
        c.start()
    for c in copies:
        c.wait()


def kernel(x, table):
    T = x.shape[1]
    _, d = table.shape
    return pl.pallas_call(
        _dma_copy,
        in_specs=[pl.BlockSpec(memory_space=pltpu.MemorySpace.HBM)],
        out_specs=pl.BlockSpec(memory_space=pltpu.MemorySpace.HBM),
        scratch_shapes=[pltpu.SemaphoreType.DMA((_N_SPLIT,))],
        out_shape=jax.ShapeDtypeStruct((T, d), table.dtype),
    )(table)


# speedup vs baseline: 47.5329x; 47.5329x over previous
import jax
import jax.numpy as jnp
from jax.experimental import pallas as pl

_ROWS_PER_BLOCK = 1024


def _copy_block(t_ref, o_ref):
    o_ref[...] = t_ref[...]


def kernel(x, table):
    T = x.shape[1]
    _, d = table.shape
    grid = (T // _ROWS_PER_BLOCK,)
    return pl.pallas_call(
        _copy_block,
        grid=grid,
        in_specs=[pl.BlockSpec((_ROWS_PER_BLOCK, d), lambda i: (i, 0))],
        out_specs=pl.BlockSpec((_ROWS_PER_BLOCK, d), lambda i: (i, 0)),
        out_shape=jax.ShapeDtypeStruct((T, d), table.dtype),
    )(table)
